# baseline (speedup 1.0000x reference)
import jax
import jax.numpy as jnp
from jax import lax
from jax.experimental import pallas as pl
from jax.experimental.pallas import tpu as pltpu
from jax.experimental.pallas import tpu_sc as plsc

_SDF_MIN = -7.0
_SDF_MAX = 7.0
_N_BINS = 256

_NC = 2
_NS = 16
_NW = _NC * _NS
_L = 16

_N = 8 * 128 * 128 * 128
_PER_W = _N // _NW
_CHUNK = 16384
_NCHUNKS = _PER_W // _CHUNK
_UNROLL = 8

_A = (_N_BINS - 1) / (_SDF_MAX - _SDF_MIN)
_B = -_SDF_MIN * _A + 0.5


def _compute_chunk(yp_v, yt_v, bins_v, lane):
    @plsc.parallel_loop(0, _CHUNK, step=_UNROLL * _L)
    def body(off):
        for j in range(_UNROLL):
            t = yt_v[pl.ds(off + j * _L, _L)]
            p = yp_v[pl.ds(off + j * _L, _L)]
            tc = jnp.minimum(jnp.maximum(t, _SDF_MIN), _SDF_MAX)
            x = tc * _A + _B
            idx = x.astype(jnp.int32)
            addr = (idx << 4) + lane
            d = p - t
            plsc.addupdate_scatter(bins_v, [addr], d * d)

    return body


def _sc_body(yp_hbm, yt_hbm, lut_hbm, out_hbm,
             lut_v, bins_v, yp0, yp1, yt0, yt1, acc_v,
             sp0, sp1, st0, st1):
    c = lax.axis_index("c")
    s = lax.axis_index("s")
    wid = s * _NC + c
    base = wid * _PER_W
    pltpu.sync_copy(lut_hbm, lut_v)
    lane = lax.iota(jnp.int32, _L)

    @plsc.parallel_loop(0, _N_BINS * _L, step=_L)
    def _zero(off):
        bins_v[pl.ds(off, _L)] = jnp.zeros((_L,), jnp.float32)

    bufs = ((yp0, yt0, sp0, st0), (yp1, yt1, sp1, st1))

    def start(k, parity):
        ypb, ytb, sp, st = bufs[parity]
        off = base + k * _CHUNK
        pltpu.async_copy(yp_hbm.at[pl.ds(off, _CHUNK)], ypb, sp)
        pltpu.async_copy(yt_hbm.at[pl.ds(off, _CHUNK)], ytb, st)

    def wait(parity):
        ypb, ytb, sp, st = bufs[parity]
        pltpu.make_async_copy(yp_hbm.at[pl.ds(base, _CHUNK)], ypb, sp).wait()
        pltpu.make_async_copy(yt_hbm.at[pl.ds(base, _CHUNK)], ytb, st).wait()

    start(0, 0)
    start(1, 1)

    def pair_body(g, carry):
        k0 = 2 * g
        wait(0)
        _compute_chunk(bufs[0][0], bufs[0][1], bins_v, lane)
        start(k0 + 2, 0)
        wait(1)
        _compute_chunk(bufs[1][0], bufs[1][1], bins_v, lane)
        start(k0 + 3, 1)
        return carry

    lax.fori_loop(0, _NCHUNKS // 2 - 1, pair_body, jnp.int32(0))

    wait(0)
    _compute_chunk(bufs[0][0], bufs[0][1], bins_v, lane)
    wait(1)
    _compute_chunk(bufs[1][0], bufs[1][1], bins_v, lane)

    def fin_body(g, acc):
        wvec = lut_v[pl.ds(g * _L, _L)]
        for k in range(_L):
            row = bins_v[pl.ds(g * _L * _L + k * _L, _L)]
            acc = acc + row * wvec[k]
        return acc

    acc = lax.fori_loop(0, _N_BINS // _L, fin_body, jnp.zeros((_L,), jnp.float32))
    acc_v[...] = acc
    pltpu.sync_copy(acc_v, out_hbm.at[wid])


@jax.jit
def _sc_partials(yp, yt, lut):
    mesh = plsc.VectorSubcoreMesh(core_axis_name="c", subcore_axis_name="s")
    return pl.kernel(
        _sc_body,
        out_type=jax.ShapeDtypeStruct((_NW, _L), jnp.float32),
        mesh=mesh,
        scratch_types=[
            pltpu.VMEM((_N_BINS,), jnp.float32),
            pltpu.VMEM((_N_BINS * _L,), jnp.float32),
            pltpu.VMEM((_CHUNK,), jnp.float32),
            pltpu.VMEM((_CHUNK,), jnp.float32),
            pltpu.VMEM((_CHUNK,), jnp.float32),
            pltpu.VMEM((_CHUNK,), jnp.float32),
            pltpu.VMEM((_L,), jnp.float32),
            pltpu.SemaphoreType.DMA,
            pltpu.SemaphoreType.DMA,
            pltpu.SemaphoreType.DMA,
            pltpu.SemaphoreType.DMA,
        ],
        compiler_params=pltpu.CompilerParams(needs_layout_passes=False),
    )(yp, yt, lut)


def kernel(y_pred, y_true, lut):
    n = y_pred.size
    partials = _sc_partials(y_pred.reshape(-1), y_true.reshape(-1), lut)
    return (partials.sum() / n).astype(jnp.float32)
